# Initial kernel scaffold; baseline (speedup 1.0000x reference)
import jax
import jax.numpy as jnp
from jax.experimental import pallas as pl


def kernel(node_embeddings1, node_embeddings2, temperature, fusion_weights):
    raise NotImplementedError("write your pallas kernel here")


# speedup vs baseline: 8.1781x; 8.1781x over previous
import functools

import jax
import jax.numpy as jnp
from jax.experimental import pallas as pl
from jax.experimental.pallas import tpu as pltpu

_H = 4
_LANE = 128
_BITS_ONE_PLUS = 0x3F800001
_SEARCH_STEPS = 30


def _exclusive_prefix(tie_f, rows, cols):
    chunks = cols // _LANE
    t3 = tie_f.reshape(rows * chunks, _LANE)
    li = jax.lax.broadcasted_iota(jnp.int32, (_LANE, _LANE), 0)
    lj = jax.lax.broadcasted_iota(jnp.int32, (_LANE, _LANE), 1)
    m = (li < lj).astype(jnp.float32)
    within = jnp.dot(t3, m, preferred_element_type=jnp.float32)
    within = within.reshape(rows, chunks, _LANE)
    csum = jnp.sum(t3.reshape(rows, chunks, _LANE), axis=2)
    inc = csum
    sh = 1
    while sh < chunks:
        inc = inc + jnp.concatenate(
            [jnp.zeros((rows, sh), jnp.float32), inc[:, :-sh]], axis=1
        )
        sh *= 2
    excl = inc - csum
    prefix = within + excl[:, :, None]
    return prefix.reshape(rows, cols)


def _block_body(fw_ref, invt_ref, e1_ref, e2_ref, out_ref, *, topk):
    rows, cols = out_ref.shape
    kf = jnp.float32(topk)
    acc = jnp.zeros((rows, cols), jnp.float32)
    for h in range(_H):
        x = jnp.dot(e1_ref[h], e2_ref[h], preferred_element_type=jnp.float32)
        x = jnp.maximum(x, 0.0) * invt_ref[h]
        m = jnp.max(x, axis=1, keepdims=True)
        e = jnp.exp(x - m)
        z = jnp.sum(e, axis=1, keepdims=True)
        p = e / z
        pb = jax.lax.bitcast_convert_type(p, jnp.int32)

        def _bisect(_, carry):
            lo, hi = carry
            mid = lo + jax.lax.shift_right_logical(hi - lo, 1)
            cnt = jnp.sum((pb >= mid).astype(jnp.float32), axis=1, keepdims=True)
            ok = cnt >= kf
            return jnp.where(ok, mid, lo), jnp.where(ok, hi, mid)

        lo0 = jnp.zeros((rows, 1), jnp.int32)
        hi0 = jnp.full((rows, 1), _BITS_ONE_PLUS, jnp.int32)
        tb, _ = jax.lax.fori_loop(0, _SEARCH_STEPS, _bisect, (lo0, hi0))

        gt = pb > tb
        tie = pb == tb
        c_gt = jnp.sum(gt.astype(jnp.float32), axis=1, keepdims=True)
        k_rem = kf - c_gt
        prefix = _exclusive_prefix(tie.astype(jnp.float32), rows, cols)
        sel_mask = gt | (tie & (prefix < k_rem))

        sel = jnp.where(sel_mask, p, 0.0)
        s = jnp.sum(sel, axis=1, keepdims=True)
        acc = acc + sel * (fw_ref[h] / (s + 1e-8))
    out_ref[...] = acc


def _fused_topk_adj(e1, e2, fw, invt, *, topk, block_rows):
    h, n, d = e1.shape
    grid = (n // block_rows,)
    return pl.pallas_call(
        functools.partial(_block_body, topk=topk),
        grid=grid,
        in_specs=[
            pl.BlockSpec(memory_space=pltpu.SMEM),
            pl.BlockSpec(memory_space=pltpu.SMEM),
            pl.BlockSpec((h, block_rows, d), lambda i: (0, i, 0)),
            pl.BlockSpec((h, d, n), lambda i: (0, 0, 0)),
        ],
        out_specs=pl.BlockSpec((block_rows, n), lambda i: (i, 0)),
        out_shape=jax.ShapeDtypeStruct((n, n), jnp.float32),
    )(fw, invt, e1, e2)


def kernel(node_embeddings1, node_embeddings2, temperature, fusion_weights):
    temp = jnp.clip(temperature, 0.1, 2.0)
    invt = 1.0 / temp
    fw = jax.nn.softmax(fusion_weights, axis=0)
    return _fused_topk_adj(
        node_embeddings1,
        node_embeddings2,
        fw,
        invt,
        topk=32,
        block_rows=256,
    )


# speedup vs baseline: 9.9957x; 1.2223x over previous
import functools

import jax
import jax.numpy as jnp
from jax.experimental import pallas as pl
from jax.experimental.pallas import tpu as pltpu

_H = 4
_LANE = 128
_BITS_ONE_PLUS = 0x3F800001
_SEARCH_STEPS = 30


def _exclusive_prefix(tie_f, rows, cols):
    chunks = cols // _LANE
    t3 = tie_f.reshape(rows * chunks, _LANE)
    li = jax.lax.broadcasted_iota(jnp.int32, (_LANE, _LANE), 0)
    lj = jax.lax.broadcasted_iota(jnp.int32, (_LANE, _LANE), 1)
    m = (li < lj).astype(jnp.float32)
    within = jnp.dot(t3, m, preferred_element_type=jnp.float32)
    within = within.reshape(rows, chunks, _LANE)
    csum = jnp.sum(t3.reshape(rows, chunks, _LANE), axis=2)
    inc = csum
    sh = 1
    while sh < chunks:
        inc = inc + jnp.concatenate(
            [jnp.zeros((rows, sh), jnp.float32), inc[:, :-sh]], axis=1
        )
        sh *= 2
    excl = inc - csum
    prefix = within + excl[:, :, None]
    return prefix.reshape(rows, cols)


def _block_body(fw_ref, invt_ref, e1_ref, e2_ref, out_ref, *, topk):
    rows, cols = out_ref.shape
    kf = jnp.float32(topk)
    acc = jnp.zeros((rows, cols), jnp.float32)
    for h in range(_H):
        x = jnp.dot(e1_ref[h], e2_ref[h], preferred_element_type=jnp.float32)
        x = jnp.maximum(x, 0.0) * invt_ref[h]
        m = jnp.max(x, axis=1, keepdims=True)
        e = jnp.exp(x - m)
        z = jnp.sum(e, axis=1, keepdims=True)
        p = e / z
        pb = jax.lax.bitcast_convert_type(p, jnp.int32)

        chunks = cols // _LANE
        if chunks >= topk:
            cmax = jnp.max(p.reshape(rows, chunks, _LANE), axis=2)
            lo0 = jax.lax.bitcast_convert_type(
                jnp.min(cmax, axis=1, keepdims=True), jnp.int32
            )
            hi0 = (
                jax.lax.bitcast_convert_type(
                    jnp.max(cmax, axis=1, keepdims=True), jnp.int32
                )
                + 1
            )
        else:
            lo0 = jnp.zeros((rows, 1), jnp.int32)
            hi0 = jnp.full((rows, 1), _BITS_ONE_PLUS, jnp.int32)

        def _cond(carry):
            _, _, cont = carry
            return cont > 0

        def _bisect(carry):
            lo, hi, _ = carry
            mid = lo + jax.lax.shift_right_logical(hi - lo, 1)
            cnt = jnp.sum((pb >= mid).astype(jnp.float32), axis=1, keepdims=True)
            ok = cnt >= kf
            lo = jnp.where(ok, mid, lo)
            hi = jnp.where(ok, hi, mid)
            cont = jnp.max(hi - lo).astype(jnp.int32) > 1
            return lo, hi, cont.astype(jnp.int32)

        tb, _, _ = jax.lax.while_loop(
            _cond, _bisect, (lo0, hi0, jnp.int32(1))
        )

        gt = pb > tb
        tie = pb == tb
        c_gt = jnp.sum(gt.astype(jnp.float32), axis=1, keepdims=True)
        k_rem = kf - c_gt
        prefix = _exclusive_prefix(tie.astype(jnp.float32), rows, cols)
        sel_mask = gt | (tie & (prefix < k_rem))

        sel = jnp.where(sel_mask, p, 0.0)
        s = jnp.sum(sel, axis=1, keepdims=True)
        acc = acc + sel * (fw_ref[h] / (s + 1e-8))
    out_ref[...] = acc


def _fused_topk_adj(e1, e2, fw, invt, *, topk, block_rows):
    h, n, d = e1.shape
    grid = (n // block_rows,)
    return pl.pallas_call(
        functools.partial(_block_body, topk=topk),
        grid=grid,
        in_specs=[
            pl.BlockSpec(memory_space=pltpu.SMEM),
            pl.BlockSpec(memory_space=pltpu.SMEM),
            pl.BlockSpec((h, block_rows, d), lambda i: (0, i, 0)),
            pl.BlockSpec((h, d, n), lambda i: (0, 0, 0)),
        ],
        out_specs=pl.BlockSpec((block_rows, n), lambda i: (i, 0)),
        out_shape=jax.ShapeDtypeStruct((n, n), jnp.float32),
    )(fw, invt, e1, e2)


def kernel(node_embeddings1, node_embeddings2, temperature, fusion_weights):
    temp = jnp.clip(temperature, 0.1, 2.0)
    invt = 1.0 / temp
    fw = jax.nn.softmax(fusion_weights, axis=0)
    return _fused_topk_adj(
        node_embeddings1,
        node_embeddings2,
        fw,
        invt,
        topk=32,
        block_rows=256,
    )
